# Initial kernel scaffold; baseline (speedup 1.0000x reference)
import functools

import jax
import jax.numpy as jnp
from jax import lax
from jax.experimental import pallas as pl
from jax.experimental.pallas import tpu as pltpu
from jax.experimental.pallas import tpu_sc as plsc

NSTU, NDORM, NCOURSE = 100000, 25000, 2000
DIN, DHID, DOUT, NH = 128, 128, 64, 32
ACC_STU, ACC_DORM, ACC_COURSE = 100352, 25088, 2048
NC, NS = 2, 16
SB = 128
NB = 5
EP_LIVE, EP_CHOOSE = 102400, 409600


def _dma_idx(r):
    return r


def _scatter_add(src_ref, acc_ref, idx_ref):
    pltpu.sync_copy(src_ref, acc_ref.at[_dma_idx(idx_ref)], add=True)


def _core_id():
    return lax.axis_index("c")


def _subcore_id():
    return lax.axis_index("s")


def _divisor_le(n, cap):
    for d in range(min(n, cap), 0, -1):
        if n % d == 0:
            return d
    return 1


def _make_sc_group(phases, tab_shapes, acc_shape, out_shapes, interpret=False):
    mesh = plsc.VectorSubcoreMesh(core_axis_name="c", subcore_axis_name="s",
                                  num_cores=NC, num_subcores=NS)
    max_nsb = max(p["e_pad"] // (NC * NS) // SB for p in phases)
    max_dc = max(p["dc"] for p in phases)
    zb_rows = 1
    for p in phases:
        rpt = p["n_acc"] // NS
        zb_rows = max(zb_rows, _divisor_le(rpt, 102400 // (4 * p["dc"])))
    n_tabs = len(tab_shapes)
    n_outs = len(out_shapes)

    scratch = [
        pltpu.VMEM((max_nsb, SB), jnp.int32),
        pltpu.VMEM((max_nsb, SB), jnp.int32),
        pltpu.VMEM((NB, SB), jnp.int32),
        pltpu.VMEM((NB, SB, max_dc), jnp.float32),
        pltpu.VMEM((zb_rows, max_dc), jnp.float32),
        pltpu.VMEM_SHARED(acc_shape, jnp.float32),
        pltpu.SemaphoreType.DMA,
    ]

    @functools.partial(
        pl.kernel, mesh=mesh,
        out_type=[jax.ShapeDtypeStruct(s, jnp.float32) for s in out_shapes],
        scratch_types=scratch,
        interpret=interpret,
    )
    def k(*refs):
        tabs = refs[:n_tabs]
        idx_refs = refs[n_tabs:n_tabs + 4]
        outs = refs[n_tabs + 4:n_tabs + 4 + n_outs]
        didx_v, sidx_v, gidx_v, rows_v, zb_v, acc_sh, sem = \
            refs[n_tabs + 4 + n_outs:]
        c = _core_id()
        s = _subcore_id()
        tile = c * NS + s

        zero16 = jnp.zeros((16,), jnp.float32)
        one16 = jnp.ones((16,), jnp.float32)

        def zfill(z, carry):
            for j in range(max_dc // 16):
                zb_v[z, pl.ds(j * 16, 16)] = zero16
            return carry
        lax.fori_loop(0, zb_rows, zfill, 0, unroll=False)

        if any(ph["tab"] is None for ph in phases):
            def ofill(z, carry):
                b = z // SB
                r = z % SB
                for j in range(max_dc // 16):
                    rows_v[b, r, pl.ds(j * 16, 16)] = one16
                return carry
            lax.fori_loop(0, NB * SB, ofill, 0, unroll=False)

        for ph in phases:
            e_pad, Dc, P = ph["e_pad"], ph["dc"], ph["p"]
            n_acc = ph["n_acc"]
            nt = e_pad // (NC * NS)
            n_sb = nt // SB
            n_grp = n_sb // NB
            rpt = n_acc // NS
            zb = _divisor_le(rpt, 102400 // (4 * Dc))
            gather = ph["tab"] is not None
            tab = tabs[ph["tab"]] if gather else None
            src_hbm = idx_refs[ph["src"]] if gather else None
            dst_hbm = idx_refs[ph["dst"]]
            out = outs[ph["out"]]
            ebase = tile * nt

            def idx_stage(i, carry):
                pltpu.sync_copy(dst_hbm.at[pl.ds(ebase + i * SB, SB)],
                                didx_v.at[i])
                if gather:
                    pltpu.sync_copy(src_hbm.at[pl.ds(ebase + i * SB, SB)],
                                    sidx_v.at[i])
                return carry
            lax.fori_loop(0, n_sb, idx_stage, 0, unroll=False)

            def pass_body(p_i, carry):
                def zloop(z, cz):
                    pltpu.sync_copy(
                        zb_v.at[pl.ds(0, zb), pl.ds(0, Dc)],
                        acc_sh.at[pl.ds(s * rpt + z * zb, zb), pl.ds(0, Dc)])
                    return cz
                lax.fori_loop(0, rpt // zb, zloop, 0, unroll=False)
                plsc.subcore_barrier()

                def gloop(g, cg):
                    descs = []
                    for b in range(NB):
                        j = g * NB + b
                        if gather:
                            if P > 1:
                                for q in range(SB // 16):
                                    raw = sidx_v[j, pl.ds(q * 16, 16)]
                                    gidx_v[b, pl.ds(q * 16, 16)] = (
                                        raw * P + p_i)
                                gi = gidx_v.at[b]
                            else:
                                gi = sidx_v.at[j]
                            descs.append(pltpu.async_copy(
                                tab.at[_dma_idx(gi)],
                                rows_v.at[b, pl.ds(0, SB), pl.ds(0, Dc)],
                                sem))
                    for b in range(NB):
                        j = g * NB + b
                        if gather:
                            descs[b].wait()
                        _scatter_add(rows_v.at[b, pl.ds(0, SB), pl.ds(0, Dc)],
                                     acc_sh.at[pl.ds(0, n_acc), pl.ds(0, Dc)],
                                     didx_v.at[j])
                    return cg
                lax.fori_loop(0, n_grp, gloop, 0, unroll=False)
                plsc.subcore_barrier()
                pltpu.sync_copy(
                    acc_sh.at[pl.ds(s * rpt, rpt), pl.ds(0, Dc)],
                    out.at[c, pl.ds(s * rpt, rpt), pl.ds(p_i * Dc, Dc)])
                return carry

            lax.fori_loop(0, P, pass_body, 0, unroll=False)
            plsc.subcore_barrier()

    return k


@functools.cache
def _sc_groups(interpret=False):
    g = {}
    live = dict(e_pad=EP_LIVE)
    ch = dict(e_pad=EP_CHOOSE)
    g["deg"] = _make_sc_group(
        phases=[
            dict(**live, src=None, dst=0, tab=None, n_acc=ACC_STU, dc=16,
                 p=1, out=0),
            dict(**live, src=None, dst=1, tab=None, n_acc=ACC_DORM, dc=16,
                 p=1, out=1),
            dict(**ch, src=None, dst=2, tab=None, n_acc=ACC_STU, dc=16,
                 p=1, out=2),
            dict(**ch, src=None, dst=3, tab=None, n_acc=ACC_COURSE, dc=16,
                 p=1, out=3),
        ],
        tab_shapes=[], acc_shape=(ACC_STU, 16),
        out_shapes=[(NC, ACC_STU, 16), (NC, ACC_DORM, 16),
                    (NC, ACC_STU, 16), (NC, ACC_COURSE, 16)],
        interpret=interpret)
    g["l1fwd"] = _make_sc_group(
        phases=[
            dict(**live, src=0, dst=1, tab=0, n_acc=ACC_DORM, dc=64, p=2,
                 out=0),
            dict(**ch, src=2, dst=3, tab=1, n_acc=ACC_COURSE, dc=64, p=2,
                 out=1),
        ],
        tab_shapes=[(ACC_STU * 2, 64), (ACC_STU * 2, 64)],
        acc_shape=(ACC_DORM, 64),
        out_shapes=[(NC, ACC_DORM, 128), (NC, ACC_COURSE, 128)],
        interpret=interpret)
    g["l1rev"] = _make_sc_group(
        phases=[
            dict(**live, src=1, dst=0, tab=0, n_acc=ACC_STU, dc=16, p=8,
                 out=0),
            dict(**ch, src=3, dst=2, tab=1, n_acc=ACC_STU, dc=16, p=8,
                 out=1),
        ],
        tab_shapes=[(ACC_DORM * 8, 16), (ACC_COURSE * 8, 16)],
        acc_shape=(ACC_STU, 16),
        out_shapes=[(NC, ACC_STU, 128), (NC, ACC_STU, 128)],
        interpret=interpret)
    g["l2fwd"] = _make_sc_group(
        phases=[
            dict(**live, src=0, dst=1, tab=0, n_acc=ACC_DORM, dc=64, p=1,
                 out=0),
            dict(**ch, src=2, dst=3, tab=1, n_acc=ACC_COURSE, dc=64, p=1,
                 out=1),
        ],
        tab_shapes=[(ACC_STU, 64), (ACC_STU, 64)],
        acc_shape=(ACC_DORM, 64),
        out_shapes=[(NC, ACC_DORM, 64), (NC, ACC_COURSE, 64)],
        interpret=interpret)
    g["l2rev"] = _make_sc_group(
        phases=[
            dict(**live, src=1, dst=0, tab=0, n_acc=ACC_STU, dc=16, p=4,
                 out=0),
            dict(**ch, src=3, dst=2, tab=1, n_acc=ACC_STU, dc=16, p=4,
                 out=1),
        ],
        tab_shapes=[(ACC_DORM * 4, 16), (ACC_COURSE * 4, 16)],
        acc_shape=(ACC_STU, 16),
        out_shapes=[(NC, ACC_STU, 64), (NC, ACC_STU, 64)],
        interpret=interpret)
    return g


_R = 512


def _norm(d):
    return jnp.where(d > 0, lax.rsqrt(jnp.maximum(d, 1.0)), 0.0)


def _tc_call(body, grid, in_specs, out_shapes, interpret=False):
    return pl.pallas_call(
        body, grid=grid, in_specs=in_specs,
        out_specs=[pl.BlockSpec((_R,) + s[1:],
                                lambda i, _n=len(s) - 1: (i,) + (0,) * _n)
                   for s in out_shapes],
        out_shape=[jax.ShapeDtypeStruct(s, jnp.float32) for s in out_shapes],
        interpret=interpret,
    )


def _full_spec(shape):
    return pl.BlockSpec(shape, lambda i, _n=len(shape): (0,) * _n)


def _row_spec(shape):
    return pl.BlockSpec((_R,) + shape[1:],
                        lambda i, _n=len(shape) - 1: (i,) + (0,) * _n)


def _sum_spec(shape):
    return pl.BlockSpec((2, _R) + shape[2:],
                        lambda i, _n=len(shape) - 2: (0, i) + (0,) * _n)


def _tables1(stu_x, deg_sl, deg_sc, interpret=False):
    def body(x_ref, dsl_ref, dsc_ref, tl_ref, tc_ref):
        i = pl.program_id(0)
        x = x_ref[...]
        nsl = _norm(dsl_ref[pl.ds(i * _R, _R)])
        nsc = _norm(dsc_ref[pl.ds(i * _R, _R)])
        tl_ref[...] = x * nsl[:, None]
        tc_ref[...] = x * nsc[:, None]
    return _tc_call(
        body, grid=(ACC_STU // _R,),
        in_specs=[_row_spec(stu_x.shape), _full_spec(deg_sl.shape),
                  _full_spec(deg_sc.shape)],
        out_shapes=[(ACC_STU, DIN), (ACC_STU, DIN)],
        interpret=interpret)(stu_x, deg_sl, deg_sc)


def _scaled_matmul(x, deg, W, n_rows, interpret=False):
    def body(x_ref, d_ref, w_ref, o_ref):
        i = pl.program_id(0)
        n = _norm(d_ref[pl.ds(i * _R, _R)])
        o_ref[...] = jnp.dot(x_ref[...] * n[:, None], w_ref[...],
                             preferred_element_type=jnp.float32)
    return _tc_call(
        body, grid=(n_rows // _R,),
        in_specs=[_row_spec(x.shape), _full_spec(deg.shape),
                  _full_spec(W.shape)],
        out_shapes=[(n_rows, W.shape[1])],
        interpret=interpret)(x, deg, W)[0]


def _h1_to_table(A, deg, W1, b1, W2, n_rows, interpret=False):
    def body(a_ref, d_ref, w1_ref, b1_ref, w2_ref, o_ref):
        i = pl.program_id(0)
        n = _norm(d_ref[pl.ds(i * _R, _R)])
        A2 = a_ref[0] + a_ref[1]
        h = jnp.dot(A2, w1_ref[...], preferred_element_type=jnp.float32)
        h = jnp.maximum(h * n[:, None] + b1_ref[...][None, :], 0.0)
        o_ref[...] = jnp.dot(h * n[:, None], w2_ref[...],
                             preferred_element_type=jnp.float32)
    return _tc_call(
        body, grid=(n_rows // _R,),
        in_specs=[_sum_spec(A.shape), _full_spec(deg.shape),
                  _full_spec(W1.shape), _full_spec(b1.shape),
                  _full_spec(W2.shape)],
        out_shapes=[(n_rows, W2.shape[1])],
        interpret=interpret)(A, deg, W1, b1, W2)[0]


def _h1_stu_tables(Al, Ac, deg_sl, deg_sc, b1l, b1c, W2l, W2c,
                   interpret=False):
    def body(al_ref, ac_ref, dsl_ref, dsc_ref, b1l_ref, b1c_ref,
             w2l_ref, w2c_ref, fl_ref, fc_ref):
        i = pl.program_id(0)
        nsl = _norm(dsl_ref[pl.ds(i * _R, _R)])[:, None]
        nsc = _norm(dsc_ref[pl.ds(i * _R, _R)])[:, None]
        Al2 = al_ref[0] + al_ref[1]
        Ac2 = ac_ref[0] + ac_ref[1]
        h1 = jnp.maximum(
            0.5 * (nsl * Al2 + nsc * Ac2)
            + 0.5 * (b1l_ref[...] + b1c_ref[...])[None, :], 0.0)
        fl_ref[...] = jnp.dot(h1 * nsl, w2l_ref[...],
                              preferred_element_type=jnp.float32)
        fc_ref[...] = jnp.dot(h1 * nsc, w2c_ref[...],
                              preferred_element_type=jnp.float32)
    return _tc_call(
        body, grid=(ACC_STU // _R,),
        in_specs=[_sum_spec(Al.shape), _sum_spec(Ac.shape),
                  _full_spec(deg_sl.shape), _full_spec(deg_sc.shape),
                  _full_spec(b1l.shape), _full_spec(b1c.shape),
                  _full_spec(W2l.shape), _full_spec(W2c.shape)],
        out_shapes=[(ACC_STU, DOUT), (ACC_STU, DOUT)],
        interpret=interpret)(Al, Ac, deg_sl, deg_sc, b1l, b1c, W2l, W2c)


def _h2_post(B, deg, b2, n_rows, interpret=False):
    def body(b_ref, d_ref, b2_ref, o_ref):
        i = pl.program_id(0)
        n = _norm(d_ref[pl.ds(i * _R, _R)])
        o_ref[...] = (b_ref[0] + b_ref[1]) * n[:, None] + b2_ref[...][None, :]
    return _tc_call(
        body, grid=(n_rows // _R,),
        in_specs=[_sum_spec(B.shape), _full_spec(deg.shape),
                  _full_spec(b2.shape)],
        out_shapes=[(n_rows, DOUT)],
        interpret=interpret)(B, deg, b2)[0]


def _lstm_gates(g):
    i = jax.nn.sigmoid(g[:, 0:NH])
    gg = jnp.tanh(g[:, 2 * NH:3 * NH])
    o = jax.nn.sigmoid(g[:, 3 * NH:4 * NH])
    return o * jnp.tanh(i * gg)


def _stu_final(Bl, Bc, deg_sl, deg_sc, b2l, b2c, Wf, bf, Wb, bb,
               interpret=False):
    def body(bl_ref, bc_ref, dsl_ref, dsc_ref, b2l_ref, b2c_ref,
             wf_ref, bf_ref, wb_ref, bb_ref, o_ref):
        i = pl.program_id(0)
        nsl = _norm(dsl_ref[pl.ds(i * _R, _R)])[:, None]
        nsc = _norm(dsc_ref[pl.ds(i * _R, _R)])[:, None]
        h2 = 0.5 * (nsl * (bl_ref[0] + bl_ref[1])
                    + nsc * (bc_ref[0] + bc_ref[1])
                    + (b2l_ref[...] + b2c_ref[...])[None, :])
        gf = jnp.dot(h2, wf_ref[...], preferred_element_type=jnp.float32) \
            + bf_ref[...][None, :]
        gb = jnp.dot(h2, wb_ref[...], preferred_element_type=jnp.float32) \
            + bb_ref[...][None, :]
        o_ref[...] = jnp.concatenate([_lstm_gates(gf), _lstm_gates(gb)],
                                     axis=1)
    return _tc_call(
        body, grid=(ACC_STU // _R,),
        in_specs=[_sum_spec(Bl.shape), _sum_spec(Bc.shape),
                  _full_spec(deg_sl.shape), _full_spec(deg_sc.shape),
                  _full_spec(b2l.shape), _full_spec(b2c.shape),
                  _full_spec(Wf.shape), _full_spec(bf.shape),
                  _full_spec(Wb.shape), _full_spec(bb.shape)],
        out_shapes=[(ACC_STU, 2 * NH)],
        interpret=interpret)(Bl, Bc, deg_sl, deg_sc, b2l, b2c,
                             Wf, bf, Wb, bb)[0]


def _pad_idx(idx, e_pad, fill):
    idx = idx.astype(jnp.int32)
    return jnp.concatenate(
        [idx, jnp.full((e_pad - idx.shape[0],), fill, jnp.int32)])


def kernel(stu_x, dorm_x, course_x, live_src, live_dst, choose_src, choose_dst,
           W1_live, b1_live, W1_livedby, b1_livedby, W1_choose, b1_choose,
           W1_choosedby, b1_choosedby, W2_live, b2_live, W2_livedby,
           b2_livedby, W2_choose, b2_choose, W2_choosedby, b2_choosedby,
           W_ih_f, W_hh_f, b_ih_f, b_hh_f, W_ih_b, W_hh_b, b_ih_b, b_hh_b):
    interp = False
    ls = _pad_idx(live_src, EP_LIVE, NSTU)
    ld = _pad_idx(live_dst, EP_LIVE, NDORM)
    cs = _pad_idx(choose_src, EP_CHOOSE, NSTU)
    cd = _pad_idx(choose_dst, EP_CHOOSE, NCOURSE)
    idxs = (ls, ld, cs, cd)
    g = _sc_groups(interp)

    dsl, dd, dsc, dcr = g["deg"](*idxs)
    deg_sl = dsl[0, :, 0] + dsl[1, :, 0]
    deg_d = dd[0, :, 0] + dd[1, :, 0]
    deg_sc = dsc[0, :, 0] + dsc[1, :, 0]
    deg_c = dcr[0, :, 0] + dcr[1, :, 0]

    t_live, t_choose = _tables1(stu_x, deg_sl, deg_sc, interp)
    y_dorm = _scaled_matmul(dorm_x, deg_d, W1_livedby, ACC_DORM, interp)
    y_course = _scaled_matmul(course_x, deg_c, W1_choosedby, ACC_COURSE,
                              interp)

    A_dorm, A_course = g["l1fwd"](
        t_live.reshape(ACC_STU * 2, 64), t_choose.reshape(ACC_STU * 2, 64),
        *idxs)
    A_stu_l, A_stu_c = g["l1rev"](
        y_dorm.reshape(ACC_DORM * 8, 16), y_course.reshape(ACC_COURSE * 8, 16),
        *idxs)

    g_dorm = _h1_to_table(A_dorm, deg_d, W1_live, b1_live, W2_livedby,
                          ACC_DORM, interp)
    g_course = _h1_to_table(A_course, deg_c, W1_choose, b1_choose,
                            W2_choosedby, ACC_COURSE, interp)
    f_live, f_choose = _h1_stu_tables(A_stu_l, A_stu_c, deg_sl, deg_sc,
                                      b1_livedby, b1_choosedby,
                                      W2_live, W2_choose, interp)

    B_dorm, B_course = g["l2fwd"](f_live, f_choose, *idxs)
    B_stu_l, B_stu_c = g["l2rev"](
        g_dorm.reshape(ACC_DORM * 4, 16), g_course.reshape(ACC_COURSE * 4, 16),
        *idxs)

    h2_dorm = _h2_post(B_dorm, deg_d, b2_live, ACC_DORM, interp)
    h2_course = _h2_post(B_course, deg_c, b2_choose, ACC_COURSE, interp)
    stu_out = _stu_final(B_stu_l, B_stu_c, deg_sl, deg_sc,
                         b2_livedby, b2_choosedby,
                         W_ih_f, b_ih_f + b_hh_f, W_ih_b, b_ih_b + b_hh_b,
                         interp)

    return jnp.concatenate(
        [stu_out[:NSTU], h2_dorm[:NDORM], h2_course[:NCOURSE]], axis=0)


# speedup vs baseline: 1.3590x; 1.3590x over previous
import functools

import jax
import jax.numpy as jnp
from jax import lax
from jax.experimental import pallas as pl
from jax.experimental.pallas import tpu as pltpu
from jax.experimental.pallas import tpu_sc as plsc

NSTU, NDORM, NCOURSE = 100000, 25000, 2000
DIN, DHID, DOUT, NH = 128, 128, 64, 32
ACC_STU, ACC_DORM, ACC_COURSE = 100352, 25088, 2048
NC, NS = 2, 16
SB = 128
NB = 5
EP_LIVE, EP_CHOOSE = 102400, 409600


def _dma_idx(r):
    return r


def _scatter_add(src_ref, acc_ref, idx_ref):
    pltpu.sync_copy(src_ref, acc_ref.at[_dma_idx(idx_ref)], add=True)


def _core_id():
    return lax.axis_index("c")


def _subcore_id():
    return lax.axis_index("s")


def _divisor_le(n, cap):
    for d in range(min(n, cap), 0, -1):
        if n % d == 0:
            return d
    return 1


def _make_sc_group(phases, tab_shapes, acc_shape, out_shapes, interpret=False):
    mesh = plsc.VectorSubcoreMesh(core_axis_name="c", subcore_axis_name="s",
                                  num_cores=NC, num_subcores=NS)
    max_nsb = max(p["e_pad"] // (NC * NS) // SB for p in phases)
    max_dc = max(p["dc"] for p in phases)
    zb_rows = 1
    for p in phases:
        rpt = p["n_acc"] // NS
        zb_rows = max(zb_rows, _divisor_le(rpt, 102400 // (4 * p["dc"])))
    n_tabs = len(tab_shapes)
    n_outs = len(out_shapes)

    scratch = [
        pltpu.VMEM((max_nsb, SB), jnp.int32),
        pltpu.VMEM((max_nsb, SB), jnp.int32),
        pltpu.VMEM((NB, SB), jnp.int32),
        pltpu.VMEM((NB, SB, max_dc), jnp.float32),
        pltpu.VMEM((zb_rows, max_dc), jnp.float32),
        pltpu.VMEM_SHARED(acc_shape, jnp.float32),
        pltpu.SemaphoreType.DMA,
    ]

    has_deg = any(ph["tab"] is None for ph in phases)

    @functools.partial(
        pl.kernel, mesh=mesh,
        out_type=[pltpu.HBM(s, jnp.float32) for s in out_shapes],
        scratch_types=scratch,
        compiler_params=pltpu.CompilerParams(use_tc_tiling_on_sc=False),
        interpret=interpret,
    )
    def k(*refs):
        n_pre = n_tabs + 1 + (1 if has_deg else 0)
        tabs = refs[:n_tabs]
        z_in = refs[n_tabs]
        o_in = refs[n_tabs + 1] if has_deg else None
        idx_refs = refs[n_pre:n_pre + 4]
        outs = refs[n_pre + 4:n_pre + 4 + n_outs]
        didx_v, sidx_v, gidx_v, rows_v, zb_v, acc_sh, sem = \
            refs[n_pre + 4 + n_outs:]
        c = _core_id()
        s = _subcore_id()
        tile = c * NS + s

        pltpu.sync_copy(z_in, zb_v)
        if has_deg:
            for b in range(NB):
                pltpu.sync_copy(o_in, rows_v.at[b])

        for ph in phases:
            e_pad, Dc, P = ph["e_pad"], ph["dc"], ph["p"]
            n_acc = ph["n_acc"]
            nt = e_pad // (NC * NS)
            n_sb = nt // SB
            n_grp = n_sb // NB
            rpt = n_acc // NS
            zb = _divisor_le(rpt, 102400 // (4 * Dc))
            gather = ph["tab"] is not None
            tab = tabs[ph["tab"]] if gather else None
            src_hbm = idx_refs[ph["src"]] if gather else None
            dst_hbm = idx_refs[ph["dst"]]
            out = outs[ph["out"]]
            ebase = tile * nt

            def idx_stage(i, carry):
                pltpu.sync_copy(dst_hbm.at[pl.ds(ebase + i * SB, SB)],
                                didx_v.at[i])
                if gather:
                    pltpu.sync_copy(src_hbm.at[pl.ds(ebase + i * SB, SB)],
                                    sidx_v.at[i])
                return carry
            lax.fori_loop(0, n_sb, idx_stage, 0, unroll=False)

            def pass_body(p_i, carry):
                def zloop(z, cz):
                    pltpu.sync_copy(
                        zb_v.at[pl.ds(0, zb), pl.ds(0, Dc)],
                        acc_sh.at[pl.ds(s * rpt + z * zb, zb), pl.ds(0, Dc)])
                    return cz
                lax.fori_loop(0, rpt // zb, zloop, 0, unroll=False)
                plsc.subcore_barrier()

                def gloop(g, cg):
                    descs = []
                    for b in range(NB):
                        j = g * NB + b
                        if gather:
                            if P > 1:
                                for q in range(SB // 16):
                                    raw = sidx_v[j, pl.ds(q * 16, 16)]
                                    gidx_v[b, pl.ds(q * 16, 16)] = (
                                        raw * P + p_i)
                                gi = gidx_v.at[b]
                            else:
                                gi = sidx_v.at[j]
                            descs.append(pltpu.async_copy(
                                tab.at[_dma_idx(gi)],
                                rows_v.at[b, pl.ds(0, SB), pl.ds(0, Dc)],
                                sem))
                    for b in range(NB):
                        j = g * NB + b
                        if gather:
                            descs[b].wait()
                        _scatter_add(rows_v.at[b, pl.ds(0, SB), pl.ds(0, Dc)],
                                     acc_sh.at[pl.ds(0, n_acc), pl.ds(0, Dc)],
                                     didx_v.at[j])
                    return cg
                lax.fori_loop(0, n_grp, gloop, 0, unroll=False)
                plsc.subcore_barrier()
                pltpu.sync_copy(
                    acc_sh.at[pl.ds(s * rpt, rpt), pl.ds(0, Dc)],
                    out.at[c, pl.ds(s * rpt, rpt), pl.ds(p_i * Dc, Dc)])
                return carry

            lax.fori_loop(0, P, pass_body, 0, unroll=False)
            plsc.subcore_barrier()

    def call(*tabs_and_idxs):
        tabs = tabs_and_idxs[:n_tabs]
        idxs = tabs_and_idxs[n_tabs:]
        pre = [jnp.zeros((zb_rows, max_dc), jnp.float32)]
        if has_deg:
            pre.append(jnp.ones((SB, max_dc), jnp.float32))
        return k(*tabs, *pre, *idxs)

    return call


@functools.cache
def _sc_groups(interpret=False):
    g = {}
    live = dict(e_pad=EP_LIVE)
    ch = dict(e_pad=EP_CHOOSE)
    g["deg"] = _make_sc_group(
        phases=[
            dict(**live, src=None, dst=0, tab=None, n_acc=ACC_STU, dc=8,
                 p=1, out=0),
            dict(**live, src=None, dst=1, tab=None, n_acc=ACC_DORM, dc=8,
                 p=1, out=1),
            dict(**ch, src=None, dst=2, tab=None, n_acc=ACC_STU, dc=8,
                 p=1, out=2),
            dict(**ch, src=None, dst=3, tab=None, n_acc=ACC_COURSE, dc=8,
                 p=1, out=3),
        ],
        tab_shapes=[], acc_shape=(ACC_STU, 8),
        out_shapes=[(NC, ACC_STU, 8), (NC, ACC_DORM, 8),
                    (NC, ACC_STU, 8), (NC, ACC_COURSE, 8)],
        interpret=interpret)
    g["l1fwd"] = _make_sc_group(
        phases=[
            dict(**live, src=0, dst=1, tab=0, n_acc=ACC_DORM, dc=32, p=4,
                 out=0),
            dict(**ch, src=2, dst=3, tab=1, n_acc=ACC_COURSE, dc=32, p=4,
                 out=1),
        ],
        tab_shapes=[(ACC_STU * 4, 32), (ACC_STU * 4, 32)],
        acc_shape=(ACC_DORM, 32),
        out_shapes=[(NC, ACC_DORM, 128), (NC, ACC_COURSE, 128)],
        interpret=interpret)
    g["l1rev"] = _make_sc_group(
        phases=[
            dict(**live, src=1, dst=0, tab=0, n_acc=ACC_STU, dc=8, p=16,
                 out=0),
            dict(**ch, src=3, dst=2, tab=1, n_acc=ACC_STU, dc=8, p=16,
                 out=1),
        ],
        tab_shapes=[(ACC_DORM * 16, 8), (ACC_COURSE * 16, 8)],
        acc_shape=(ACC_STU, 8),
        out_shapes=[(NC, ACC_STU, 128), (NC, ACC_STU, 128)],
        interpret=interpret)
    g["l2fwd"] = _make_sc_group(
        phases=[
            dict(**live, src=0, dst=1, tab=0, n_acc=ACC_DORM, dc=32, p=2,
                 out=0),
            dict(**ch, src=2, dst=3, tab=1, n_acc=ACC_COURSE, dc=32, p=2,
                 out=1),
        ],
        tab_shapes=[(ACC_STU * 2, 32), (ACC_STU * 2, 32)],
        acc_shape=(ACC_DORM, 32),
        out_shapes=[(NC, ACC_DORM, 64), (NC, ACC_COURSE, 64)],
        interpret=interpret)
    g["l2rev"] = _make_sc_group(
        phases=[
            dict(**live, src=1, dst=0, tab=0, n_acc=ACC_STU, dc=8, p=8,
                 out=0),
            dict(**ch, src=3, dst=2, tab=1, n_acc=ACC_STU, dc=8, p=8,
                 out=1),
        ],
        tab_shapes=[(ACC_DORM * 8, 8), (ACC_COURSE * 8, 8)],
        acc_shape=(ACC_STU, 8),
        out_shapes=[(NC, ACC_STU, 64), (NC, ACC_STU, 64)],
        interpret=interpret)
    return g


_R = 512


def _norm(d):
    return jnp.where(d > 0, lax.rsqrt(jnp.maximum(d, 1.0)), 0.0)


def _tc_call(body, grid, in_specs, out_shapes, interpret=False):
    return pl.pallas_call(
        body, grid=grid, in_specs=in_specs,
        out_specs=[pl.BlockSpec((_R,) + s[1:],
                                lambda i, _n=len(s) - 1: (i,) + (0,) * _n)
                   for s in out_shapes],
        out_shape=[jax.ShapeDtypeStruct(s, jnp.float32) for s in out_shapes],
        interpret=interpret,
    )


def _full_spec(shape):
    return pl.BlockSpec(shape, lambda i, _n=len(shape): (0,) * _n)


def _row_spec(shape):
    return pl.BlockSpec((_R,) + shape[1:],
                        lambda i, _n=len(shape) - 1: (i,) + (0,) * _n)


def _sum_spec(shape):
    return pl.BlockSpec((2, _R) + shape[2:],
                        lambda i, _n=len(shape) - 2: (0, i) + (0,) * _n)


def _tables1(stu_x, deg_sl, deg_sc, interpret=False):
    def body(x_ref, dsl_ref, dsc_ref, tl_ref, tc_ref):
        i = pl.program_id(0)
        x = x_ref[...]
        nsl = _norm(dsl_ref[pl.ds(i * _R, _R)])
        nsc = _norm(dsc_ref[pl.ds(i * _R, _R)])
        tl_ref[...] = x * nsl[:, None]
        tc_ref[...] = x * nsc[:, None]
    return _tc_call(
        body, grid=(ACC_STU // _R,),
        in_specs=[_row_spec(stu_x.shape), _full_spec(deg_sl.shape),
                  _full_spec(deg_sc.shape)],
        out_shapes=[(ACC_STU, DIN), (ACC_STU, DIN)],
        interpret=interpret)(stu_x, deg_sl, deg_sc)


def _scaled_matmul(x, deg, W, n_rows, interpret=False):
    def body(x_ref, d_ref, w_ref, o_ref):
        i = pl.program_id(0)
        n = _norm(d_ref[pl.ds(i * _R, _R)])
        o_ref[...] = jnp.dot(x_ref[...] * n[:, None], w_ref[...],
                             preferred_element_type=jnp.float32)
    return _tc_call(
        body, grid=(n_rows // _R,),
        in_specs=[_row_spec(x.shape), _full_spec(deg.shape),
                  _full_spec(W.shape)],
        out_shapes=[(n_rows, W.shape[1])],
        interpret=interpret)(x, deg, W)[0]


def _h1_to_table(A, deg, W1, b1, W2, n_rows, interpret=False):
    def body(a_ref, d_ref, w1_ref, b1_ref, w2_ref, o_ref):
        i = pl.program_id(0)
        n = _norm(d_ref[pl.ds(i * _R, _R)])
        A2 = a_ref[0] + a_ref[1]
        h = jnp.dot(A2, w1_ref[...], preferred_element_type=jnp.float32)
        h = jnp.maximum(h * n[:, None] + b1_ref[...][None, :], 0.0)
        o_ref[...] = jnp.dot(h * n[:, None], w2_ref[...],
                             preferred_element_type=jnp.float32)
    return _tc_call(
        body, grid=(n_rows // _R,),
        in_specs=[_sum_spec(A.shape), _full_spec(deg.shape),
                  _full_spec(W1.shape), _full_spec(b1.shape),
                  _full_spec(W2.shape)],
        out_shapes=[(n_rows, W2.shape[1])],
        interpret=interpret)(A, deg, W1, b1, W2)[0]


def _h1_stu_tables(Al, Ac, deg_sl, deg_sc, b1l, b1c, W2l, W2c,
                   interpret=False):
    def body(al_ref, ac_ref, dsl_ref, dsc_ref, b1l_ref, b1c_ref,
             w2l_ref, w2c_ref, fl_ref, fc_ref):
        i = pl.program_id(0)
        nsl = _norm(dsl_ref[pl.ds(i * _R, _R)])[:, None]
        nsc = _norm(dsc_ref[pl.ds(i * _R, _R)])[:, None]
        Al2 = al_ref[0] + al_ref[1]
        Ac2 = ac_ref[0] + ac_ref[1]
        h1 = jnp.maximum(
            0.5 * (nsl * Al2 + nsc * Ac2)
            + 0.5 * (b1l_ref[...] + b1c_ref[...])[None, :], 0.0)
        fl_ref[...] = jnp.dot(h1 * nsl, w2l_ref[...],
                              preferred_element_type=jnp.float32)
        fc_ref[...] = jnp.dot(h1 * nsc, w2c_ref[...],
                              preferred_element_type=jnp.float32)
    return _tc_call(
        body, grid=(ACC_STU // _R,),
        in_specs=[_sum_spec(Al.shape), _sum_spec(Ac.shape),
                  _full_spec(deg_sl.shape), _full_spec(deg_sc.shape),
                  _full_spec(b1l.shape), _full_spec(b1c.shape),
                  _full_spec(W2l.shape), _full_spec(W2c.shape)],
        out_shapes=[(ACC_STU, DOUT), (ACC_STU, DOUT)],
        interpret=interpret)(Al, Ac, deg_sl, deg_sc, b1l, b1c, W2l, W2c)


def _h2_post(B, deg, b2, n_rows, interpret=False):
    def body(b_ref, d_ref, b2_ref, o_ref):
        i = pl.program_id(0)
        n = _norm(d_ref[pl.ds(i * _R, _R)])
        o_ref[...] = (b_ref[0] + b_ref[1]) * n[:, None] + b2_ref[...][None, :]
    return _tc_call(
        body, grid=(n_rows // _R,),
        in_specs=[_sum_spec(B.shape), _full_spec(deg.shape),
                  _full_spec(b2.shape)],
        out_shapes=[(n_rows, DOUT)],
        interpret=interpret)(B, deg, b2)[0]


def _lstm_gates(g):
    i = jax.nn.sigmoid(g[:, 0:NH])
    gg = jnp.tanh(g[:, 2 * NH:3 * NH])
    o = jax.nn.sigmoid(g[:, 3 * NH:4 * NH])
    return o * jnp.tanh(i * gg)


def _stu_final(Bl, Bc, deg_sl, deg_sc, b2l, b2c, Wf, bf, Wb, bb,
               interpret=False):
    def body(bl_ref, bc_ref, dsl_ref, dsc_ref, b2l_ref, b2c_ref,
             wf_ref, bf_ref, wb_ref, bb_ref, o_ref):
        i = pl.program_id(0)
        nsl = _norm(dsl_ref[pl.ds(i * _R, _R)])[:, None]
        nsc = _norm(dsc_ref[pl.ds(i * _R, _R)])[:, None]
        h2 = 0.5 * (nsl * (bl_ref[0] + bl_ref[1])
                    + nsc * (bc_ref[0] + bc_ref[1])
                    + (b2l_ref[...] + b2c_ref[...])[None, :])
        gf = jnp.dot(h2, wf_ref[...], preferred_element_type=jnp.float32) \
            + bf_ref[...][None, :]
        gb = jnp.dot(h2, wb_ref[...], preferred_element_type=jnp.float32) \
            + bb_ref[...][None, :]
        o_ref[...] = jnp.concatenate([_lstm_gates(gf), _lstm_gates(gb)],
                                     axis=1)
    return _tc_call(
        body, grid=(ACC_STU // _R,),
        in_specs=[_sum_spec(Bl.shape), _sum_spec(Bc.shape),
                  _full_spec(deg_sl.shape), _full_spec(deg_sc.shape),
                  _full_spec(b2l.shape), _full_spec(b2c.shape),
                  _full_spec(Wf.shape), _full_spec(bf.shape),
                  _full_spec(Wb.shape), _full_spec(bb.shape)],
        out_shapes=[(ACC_STU, 2 * NH)],
        interpret=interpret)(Bl, Bc, deg_sl, deg_sc, b2l, b2c,
                             Wf, bf, Wb, bb)[0]


def _pad_idx(idx, e_pad, fill):
    idx = idx.astype(jnp.int32)
    return jnp.concatenate(
        [idx, jnp.full((e_pad - idx.shape[0],), fill, jnp.int32)])


def kernel(stu_x, dorm_x, course_x, live_src, live_dst, choose_src, choose_dst,
           W1_live, b1_live, W1_livedby, b1_livedby, W1_choose, b1_choose,
           W1_choosedby, b1_choosedby, W2_live, b2_live, W2_livedby,
           b2_livedby, W2_choose, b2_choose, W2_choosedby, b2_choosedby,
           W_ih_f, W_hh_f, b_ih_f, b_hh_f, W_ih_b, W_hh_b, b_ih_b, b_hh_b):
    interp = False
    ls = _pad_idx(live_src, EP_LIVE, NSTU)
    ld = _pad_idx(live_dst, EP_LIVE, NDORM)
    cs = _pad_idx(choose_src, EP_CHOOSE, NSTU)
    cd = _pad_idx(choose_dst, EP_CHOOSE, NCOURSE)
    idxs = (ls, ld, cs, cd)
    g = _sc_groups(interp)

    dsl, dd, dsc, dcr = g["deg"](*idxs)
    deg_sl = dsl[0, :, 0] + dsl[1, :, 0]
    deg_d = dd[0, :, 0] + dd[1, :, 0]
    deg_sc = dsc[0, :, 0] + dsc[1, :, 0]
    deg_c = dcr[0, :, 0] + dcr[1, :, 0]

    t_live, t_choose = _tables1(stu_x, deg_sl, deg_sc, interp)
    y_dorm = _scaled_matmul(dorm_x, deg_d, W1_livedby, ACC_DORM, interp)
    y_course = _scaled_matmul(course_x, deg_c, W1_choosedby, ACC_COURSE,
                              interp)

    A_dorm, A_course = g["l1fwd"](
        t_live.reshape(ACC_STU * 4, 32), t_choose.reshape(ACC_STU * 4, 32),
        *idxs)
    A_stu_l, A_stu_c = g["l1rev"](
        y_dorm.reshape(ACC_DORM * 16, 8), y_course.reshape(ACC_COURSE * 16, 8),
        *idxs)

    g_dorm = _h1_to_table(A_dorm, deg_d, W1_live, b1_live, W2_livedby,
                          ACC_DORM, interp)
    g_course = _h1_to_table(A_course, deg_c, W1_choose, b1_choose,
                            W2_choosedby, ACC_COURSE, interp)
    f_live, f_choose = _h1_stu_tables(A_stu_l, A_stu_c, deg_sl, deg_sc,
                                      b1_livedby, b1_choosedby,
                                      W2_live, W2_choose, interp)

    B_dorm, B_course = g["l2fwd"](
        f_live.reshape(ACC_STU * 2, 32), f_choose.reshape(ACC_STU * 2, 32),
        *idxs)
    B_stu_l, B_stu_c = g["l2rev"](
        g_dorm.reshape(ACC_DORM * 8, 8), g_course.reshape(ACC_COURSE * 8, 8),
        *idxs)

    h2_dorm = _h2_post(B_dorm, deg_d, b2_live, ACC_DORM, interp)
    h2_course = _h2_post(B_course, deg_c, b2_choose, ACC_COURSE, interp)
    stu_out = _stu_final(B_stu_l, B_stu_c, deg_sl, deg_sc,
                         b2_livedby, b2_choosedby,
                         W_ih_f, b_ih_f + b_hh_f, W_ih_b, b_ih_b + b_hh_b,
                         interp)

    return jnp.concatenate(
        [stu_out[:NSTU], h2_dorm[:NDORM], h2_course[:NCOURSE]], axis=0)


# speedup vs baseline: 1.3735x; 1.0106x over previous
import functools

import jax
import jax.numpy as jnp
from jax import lax
from jax.experimental import pallas as pl
from jax.experimental.pallas import tpu as pltpu
from jax.experimental.pallas import tpu_sc as plsc

NSTU, NDORM, NCOURSE = 100000, 25000, 2000
DIN, DHID, DOUT, NH = 128, 128, 64, 32
ACC_STU, ACC_DORM, ACC_COURSE = 100352, 25088, 2048
NC, NS = 2, 16
SB = 128
NB = 5
EP_LIVE, EP_CHOOSE = 102400, 409600


def _core_id():
    return lax.axis_index("c")


def _subcore_id():
    return lax.axis_index("s")


def _divisor_le(n, cap):
    for d in range(min(n, cap), 0, -1):
        if n % d == 0:
            return d
    return 1


def _make_sc_group(phases, tab_shapes, acc_shape, out_shapes, nb=NB,
                   interpret=False):
    mesh = plsc.VectorSubcoreMesh(core_axis_name="c", subcore_axis_name="s",
                                  num_cores=NC, num_subcores=NS)
    max_nsb = max(p["e_pad"] // NS // SB for p in phases)
    max_dc = max(p["dc"] for p in phases)
    zb_rows = 1
    for p in phases:
        rpt = p["n_acc"] // NC // NS
        zb_rows = max(zb_rows, _divisor_le(rpt, 51200 // (4 * p["dc"])))
    n_tabs = len(tab_shapes)
    n_outs = len(out_shapes)
    has_deg = any(ph["tab"] is None for ph in phases)

    scratch = [
        pltpu.VMEM((max_nsb, SB), jnp.int32),
        pltpu.VMEM((max_nsb, SB), jnp.int32),
        pltpu.VMEM((nb, SB), jnp.int32),
        pltpu.VMEM((nb, SB, max_dc), jnp.float32),
        pltpu.VMEM((zb_rows, max_dc), jnp.float32),
        pltpu.VMEM_SHARED(acc_shape, jnp.float32),
        pltpu.SemaphoreType.DMA,
    ]

    @functools.partial(
        pl.kernel, mesh=mesh,
        out_type=[pltpu.HBM(s, jnp.float32) for s in out_shapes],
        scratch_types=scratch,
        compiler_params=pltpu.CompilerParams(use_tc_tiling_on_sc=False),
        interpret=interpret,
    )
    def k(*refs):
        n_pre = n_tabs + 1 + (1 if has_deg else 0)
        tabs = refs[:n_tabs]
        z_in = refs[n_tabs]
        o_in = refs[n_tabs + 1] if has_deg else None
        idx_refs = refs[n_pre:n_pre + 4]
        outs = refs[n_pre + 4:n_pre + 4 + n_outs]
        didx_v, sidx_v, gidx_v, rows_v, zb_v, acc_sh, sem = \
            refs[n_pre + 4 + n_outs:]
        c = _core_id()
        s = _subcore_id()

        pltpu.sync_copy(z_in, zb_v)
        if has_deg:
            for b in range(nb):
                pltpu.sync_copy(o_in, rows_v.at[b])

        for ph in phases:
            e_pad, Dc, P = ph["e_pad"], ph["dc"], ph["p"]
            n_acc = ph["n_acc"]
            half = n_acc // NC
            lo = c * half
            nt = e_pad // NS
            n_sb = nt // SB
            n_grp = n_sb // nb
            rpt = half // NS
            zb = _divisor_le(rpt, 51200 // (4 * Dc))
            gather = ph["tab"] is not None
            tab = tabs[ph["tab"]] if gather else None
            src_hbm = idx_refs[ph["src"]] if gather else None
            dst_hbm = idx_refs[ph["dst"]]
            out = outs[ph["out"]]
            ebase = s * nt

            def idx_stage(i, carry):
                pltpu.sync_copy(dst_hbm.at[pl.ds(ebase + i * SB, SB)],
                                didx_v.at[i])
                if gather:
                    pltpu.sync_copy(src_hbm.at[pl.ds(ebase + i * SB, SB)],
                                    sidx_v.at[i])
                for q in range(SB // 16):
                    d16 = didx_v[i, pl.ds(q * 16, 16)]
                    inr = jnp.logical_and(d16 >= lo, d16 < lo + half)
                    didx_v[i, pl.ds(q * 16, 16)] = jnp.where(
                        inr, d16 - lo, -1)
                return carry
            lax.fori_loop(0, n_sb, idx_stage, 0, unroll=False)

            def pass_body(p_i, carry):
                def zloop(z, cz):
                    pltpu.sync_copy(
                        zb_v.at[pl.ds(0, zb), pl.ds(0, Dc)],
                        acc_sh.at[pl.ds(s * rpt + z * zb, zb), pl.ds(0, Dc)])
                    return cz
                lax.fori_loop(0, rpt // zb, zloop, 0, unroll=False)
                plsc.subcore_barrier()

                def gloop(g, cg):
                    descs = []
                    for b in range(nb):
                        j = g * nb + b
                        if gather:
                            for q in range(SB // 16):
                                d16 = didx_v[j, pl.ds(q * 16, 16)]
                                r16 = sidx_v[j, pl.ds(q * 16, 16)] * P + p_i
                                gidx_v[b, pl.ds(q * 16, 16)] = jnp.where(
                                    d16 >= 0, r16, -1)
                            descs.append(pltpu.async_copy(
                                tab.at[plsc.Indices(gidx_v.at[b],
                                                    ignored_value=-1)],
                                rows_v.at[b, pl.ds(0, SB), pl.ds(0, Dc)],
                                sem))
                    for b in range(nb):
                        j = g * nb + b
                        if gather:
                            descs[b].wait()
                        pltpu.sync_copy(
                            rows_v.at[b, pl.ds(0, SB), pl.ds(0, Dc)],
                            acc_sh.at[pl.ds(0, half), pl.ds(0, Dc)].at[
                                plsc.Indices(didx_v.at[j], ignored_value=-1)],
                            add=True)
                    return cg
                lax.fori_loop(0, n_grp, gloop, 0, unroll=False)
                plsc.subcore_barrier()
                pltpu.sync_copy(
                    acc_sh.at[pl.ds(s * rpt, rpt), pl.ds(0, Dc)],
                    out.at[c, p_i, pl.ds(s * rpt, rpt)])
                return carry

            lax.fori_loop(0, P, pass_body, 0, unroll=False)
            plsc.subcore_barrier()

    def call(*tabs_and_idxs):
        tabs = tabs_and_idxs[:n_tabs]
        idxs = tabs_and_idxs[n_tabs:]
        pre = [jnp.zeros((zb_rows, max_dc), jnp.float32)]
        if has_deg:
            pre.append(jnp.ones((SB, max_dc), jnp.float32))
        return k(*tabs, *pre, *idxs)

    return call


@functools.cache
def _sc_groups(interpret=False):
    g = {}
    live = dict(e_pad=EP_LIVE)
    ch = dict(e_pad=EP_CHOOSE)
    g["deg"] = _make_sc_group(
        phases=[
            dict(**live, src=None, dst=0, tab=None, n_acc=ACC_STU, dc=8,
                 p=1, out=0),
            dict(**live, src=None, dst=1, tab=None, n_acc=ACC_DORM, dc=8,
                 p=1, out=1),
            dict(**ch, src=None, dst=2, tab=None, n_acc=ACC_STU, dc=8,
                 p=1, out=2),
            dict(**ch, src=None, dst=3, tab=None, n_acc=ACC_COURSE, dc=8,
                 p=1, out=3),
        ],
        tab_shapes=[], acc_shape=(ACC_STU // NC, 8),
        out_shapes=[(NC, 1, ACC_STU // NC, 8), (NC, 1, ACC_DORM // NC, 8),
                    (NC, 1, ACC_STU // NC, 8), (NC, 1, ACC_COURSE // NC, 8)],
        interpret=interpret)
    g["l1fwd"] = _make_sc_group(
        phases=[
            dict(**live, src=0, dst=1, tab=0, n_acc=ACC_DORM, dc=64, p=2,
                 out=0),
            dict(**ch, src=2, dst=3, tab=1, n_acc=ACC_COURSE, dc=64, p=2,
                 out=1),
        ],
        tab_shapes=[(ACC_STU * 2, 64), (ACC_STU * 2, 64)],
        acc_shape=(ACC_DORM // NC, 64),
        out_shapes=[(NC, 2, ACC_DORM // NC, 64), (NC, 2, ACC_COURSE // NC, 64)],
        nb=2, interpret=interpret)
    g["l1rev"] = _make_sc_group(
        phases=[
            dict(**live, src=1, dst=0, tab=0, n_acc=ACC_STU, dc=16, p=8,
                 out=0),
            dict(**ch, src=3, dst=2, tab=1, n_acc=ACC_STU, dc=16, p=8,
                 out=1),
        ],
        tab_shapes=[(ACC_DORM * 8, 16), (ACC_COURSE * 8, 16)],
        acc_shape=(ACC_STU // NC, 16),
        out_shapes=[(NC, 8, ACC_STU // NC, 16), (NC, 8, ACC_STU // NC, 16)],
        interpret=interpret)
    g["l2fwd"] = _make_sc_group(
        phases=[
            dict(**live, src=0, dst=1, tab=0, n_acc=ACC_DORM, dc=64, p=1,
                 out=0),
            dict(**ch, src=2, dst=3, tab=1, n_acc=ACC_COURSE, dc=64, p=1,
                 out=1),
        ],
        tab_shapes=[(ACC_STU, 64), (ACC_STU, 64)],
        acc_shape=(ACC_DORM // NC, 64),
        out_shapes=[(NC, 1, ACC_DORM // NC, 64), (NC, 1, ACC_COURSE // NC, 64)],
        nb=2, interpret=interpret)
    g["l2rev"] = _make_sc_group(
        phases=[
            dict(**live, src=1, dst=0, tab=0, n_acc=ACC_STU, dc=16, p=4,
                 out=0),
            dict(**ch, src=3, dst=2, tab=1, n_acc=ACC_STU, dc=16, p=4,
                 out=1),
        ],
        tab_shapes=[(ACC_DORM * 4, 16), (ACC_COURSE * 4, 16)],
        acc_shape=(ACC_STU // NC, 16),
        out_shapes=[(NC, 4, ACC_STU // NC, 16), (NC, 4, ACC_STU // NC, 16)],
        interpret=interpret)
    return g


_R = 512


def _norm(d):
    return jnp.where(d > 0, lax.rsqrt(jnp.maximum(d, 1.0)), 0.0)


def _tc_call(body, grid, in_specs, out_shapes, interpret=False):
    return pl.pallas_call(
        body, grid=grid, in_specs=in_specs,
        out_specs=[pl.BlockSpec((_R,) + s[1:],
                                lambda i, _n=len(s) - 1: (i,) + (0,) * _n)
                   for s in out_shapes],
        out_shape=[jax.ShapeDtypeStruct(s, jnp.float32) for s in out_shapes],
        interpret=interpret,
    )


def _full_spec(shape):
    return pl.BlockSpec(shape, lambda i, _n=len(shape): (0,) * _n)


def _row_spec(shape):
    return pl.BlockSpec((_R,) + shape[1:],
                        lambda i, _n=len(shape) - 1: (i,) + (0,) * _n)


def _tables1(stu_x, deg_sl, deg_sc, interpret=False):
    def body(x_ref, dsl_ref, dsc_ref, tl_ref, tc_ref):
        i = pl.program_id(0)
        x = x_ref[...]
        nsl = _norm(dsl_ref[pl.ds(i * _R, _R)])
        nsc = _norm(dsc_ref[pl.ds(i * _R, _R)])
        tl_ref[...] = x * nsl[:, None]
        tc_ref[...] = x * nsc[:, None]
    return _tc_call(
        body, grid=(ACC_STU // _R,),
        in_specs=[_row_spec(stu_x.shape), _full_spec(deg_sl.shape),
                  _full_spec(deg_sc.shape)],
        out_shapes=[(ACC_STU, DIN), (ACC_STU, DIN)],
        interpret=interpret)(stu_x, deg_sl, deg_sc)


def _scaled_matmul(x, deg, W, n_rows, interpret=False):
    def body(x_ref, d_ref, w_ref, o_ref):
        i = pl.program_id(0)
        n = _norm(d_ref[pl.ds(i * _R, _R)])
        o_ref[...] = jnp.dot(x_ref[...] * n[:, None], w_ref[...],
                             preferred_element_type=jnp.float32)
    return _tc_call(
        body, grid=(n_rows // _R,),
        in_specs=[_row_spec(x.shape), _full_spec(deg.shape),
                  _full_spec(W.shape)],
        out_shapes=[(n_rows, W.shape[1])],
        interpret=interpret)(x, deg, W)[0]


def _h1_to_table(A, deg, W1, b1, W2, n_rows, interpret=False):
    def body(a_ref, d_ref, w1_ref, b1_ref, w2_ref, o_ref):
        i = pl.program_id(0)
        n = _norm(d_ref[pl.ds(i * _R, _R)])
        h = jnp.dot(a_ref[...], w1_ref[...],
                    preferred_element_type=jnp.float32)
        h = jnp.maximum(h * n[:, None] + b1_ref[...][None, :], 0.0)
        o_ref[...] = jnp.dot(h * n[:, None], w2_ref[...],
                             preferred_element_type=jnp.float32)
    return _tc_call(
        body, grid=(n_rows // _R,),
        in_specs=[_row_spec(A.shape), _full_spec(deg.shape),
                  _full_spec(W1.shape), _full_spec(b1.shape),
                  _full_spec(W2.shape)],
        out_shapes=[(n_rows, W2.shape[1])],
        interpret=interpret)(A, deg, W1, b1, W2)[0]


def _h1_stu_tables(Al, Ac, deg_sl, deg_sc, b1l, b1c, W2l, W2c,
                   interpret=False):
    def body(al_ref, ac_ref, dsl_ref, dsc_ref, b1l_ref, b1c_ref,
             w2l_ref, w2c_ref, fl_ref, fc_ref):
        i = pl.program_id(0)
        nsl = _norm(dsl_ref[pl.ds(i * _R, _R)])[:, None]
        nsc = _norm(dsc_ref[pl.ds(i * _R, _R)])[:, None]
        h1 = jnp.maximum(
            0.5 * (nsl * al_ref[...] + nsc * ac_ref[...])
            + 0.5 * (b1l_ref[...] + b1c_ref[...])[None, :], 0.0)
        fl_ref[...] = jnp.dot(h1 * nsl, w2l_ref[...],
                              preferred_element_type=jnp.float32)
        fc_ref[...] = jnp.dot(h1 * nsc, w2c_ref[...],
                              preferred_element_type=jnp.float32)
    return _tc_call(
        body, grid=(ACC_STU // _R,),
        in_specs=[_row_spec(Al.shape), _row_spec(Ac.shape),
                  _full_spec(deg_sl.shape), _full_spec(deg_sc.shape),
                  _full_spec(b1l.shape), _full_spec(b1c.shape),
                  _full_spec(W2l.shape), _full_spec(W2c.shape)],
        out_shapes=[(ACC_STU, DOUT), (ACC_STU, DOUT)],
        interpret=interpret)(Al, Ac, deg_sl, deg_sc, b1l, b1c, W2l, W2c)


def _h2_post(B, deg, b2, n_rows, interpret=False):
    def body(b_ref, d_ref, b2_ref, o_ref):
        i = pl.program_id(0)
        n = _norm(d_ref[pl.ds(i * _R, _R)])
        o_ref[...] = b_ref[...] * n[:, None] + b2_ref[...][None, :]
    return _tc_call(
        body, grid=(n_rows // _R,),
        in_specs=[_row_spec(B.shape), _full_spec(deg.shape),
                  _full_spec(b2.shape)],
        out_shapes=[(n_rows, DOUT)],
        interpret=interpret)(B, deg, b2)[0]


def _lstm_gates(g):
    i = jax.nn.sigmoid(g[:, 0:NH])
    gg = jnp.tanh(g[:, 2 * NH:3 * NH])
    o = jax.nn.sigmoid(g[:, 3 * NH:4 * NH])
    return o * jnp.tanh(i * gg)


def _stu_final(Bl, Bc, deg_sl, deg_sc, b2l, b2c, Wf, bf, Wb, bb,
               interpret=False):
    def body(bl_ref, bc_ref, dsl_ref, dsc_ref, b2l_ref, b2c_ref,
             wf_ref, bf_ref, wb_ref, bb_ref, o_ref):
        i = pl.program_id(0)
        nsl = _norm(dsl_ref[pl.ds(i * _R, _R)])[:, None]
        nsc = _norm(dsc_ref[pl.ds(i * _R, _R)])[:, None]
        h2 = 0.5 * (nsl * bl_ref[...] + nsc * bc_ref[...]
                    + (b2l_ref[...] + b2c_ref[...])[None, :])
        gf = jnp.dot(h2, wf_ref[...], preferred_element_type=jnp.float32) \
            + bf_ref[...][None, :]
        gb = jnp.dot(h2, wb_ref[...], preferred_element_type=jnp.float32) \
            + bb_ref[...][None, :]
        o_ref[...] = jnp.concatenate([_lstm_gates(gf), _lstm_gates(gb)],
                                     axis=1)
    return _tc_call(
        body, grid=(ACC_STU // _R,),
        in_specs=[_row_spec(Bl.shape), _row_spec(Bc.shape),
                  _full_spec(deg_sl.shape), _full_spec(deg_sc.shape),
                  _full_spec(b2l.shape), _full_spec(b2c.shape),
                  _full_spec(Wf.shape), _full_spec(bf.shape),
                  _full_spec(Wb.shape), _full_spec(bb.shape)],
        out_shapes=[(ACC_STU, 2 * NH)],
        interpret=interpret)(Bl, Bc, deg_sl, deg_sc, b2l, b2c,
                             Wf, bf, Wb, bb)[0]


def _asm(o, n_acc, d):
    return o.transpose(0, 2, 1, 3).reshape(n_acc, d)


def _pad_idx(idx, e_pad, fill):
    idx = idx.astype(jnp.int32)
    return jnp.concatenate(
        [idx, jnp.full((e_pad - idx.shape[0],), fill, jnp.int32)])


def kernel(stu_x, dorm_x, course_x, live_src, live_dst, choose_src, choose_dst,
           W1_live, b1_live, W1_livedby, b1_livedby, W1_choose, b1_choose,
           W1_choosedby, b1_choosedby, W2_live, b2_live, W2_livedby,
           b2_livedby, W2_choose, b2_choose, W2_choosedby, b2_choosedby,
           W_ih_f, W_hh_f, b_ih_f, b_hh_f, W_ih_b, W_hh_b, b_ih_b, b_hh_b):
    interp = False
    ls = _pad_idx(live_src, EP_LIVE, NSTU)
    ld = _pad_idx(live_dst, EP_LIVE, NDORM)
    cs = _pad_idx(choose_src, EP_CHOOSE, NSTU)
    cd = _pad_idx(choose_dst, EP_CHOOSE, NCOURSE)
    idxs = (ls, ld, cs, cd)
    g = _sc_groups(interp)

    dsl, dd, dsc, dcr = g["deg"](*idxs)
    deg_sl = dsl.reshape(ACC_STU, 8)[:, 0]
    deg_d = dd.reshape(ACC_DORM, 8)[:, 0]
    deg_sc = dsc.reshape(ACC_STU, 8)[:, 0]
    deg_c = dcr.reshape(ACC_COURSE, 8)[:, 0]

    t_live, t_choose = _tables1(stu_x, deg_sl, deg_sc, interp)
    y_dorm = _scaled_matmul(dorm_x, deg_d, W1_livedby, ACC_DORM, interp)
    y_course = _scaled_matmul(course_x, deg_c, W1_choosedby, ACC_COURSE,
                              interp)

    A_dorm, A_course = g["l1fwd"](
        t_live.reshape(ACC_STU * 2, 64), t_choose.reshape(ACC_STU * 2, 64),
        *idxs)
    A_stu_l, A_stu_c = g["l1rev"](
        y_dorm.reshape(ACC_DORM * 8, 16), y_course.reshape(ACC_COURSE * 8, 16),
        *idxs)
    A_dorm = _asm(A_dorm, ACC_DORM, 128)
    A_course = _asm(A_course, ACC_COURSE, 128)
    A_stu_l = _asm(A_stu_l, ACC_STU, 128)
    A_stu_c = _asm(A_stu_c, ACC_STU, 128)

    g_dorm = _h1_to_table(A_dorm, deg_d, W1_live, b1_live, W2_livedby,
                          ACC_DORM, interp)
    g_course = _h1_to_table(A_course, deg_c, W1_choose, b1_choose,
                            W2_choosedby, ACC_COURSE, interp)
    f_live, f_choose = _h1_stu_tables(A_stu_l, A_stu_c, deg_sl, deg_sc,
                                      b1_livedby, b1_choosedby,
                                      W2_live, W2_choose, interp)

    B_dorm, B_course = g["l2fwd"](f_live, f_choose, *idxs)
    B_stu_l, B_stu_c = g["l2rev"](
        g_dorm.reshape(ACC_DORM * 4, 16), g_course.reshape(ACC_COURSE * 4, 16),
        *idxs)
    B_dorm = _asm(B_dorm, ACC_DORM, 64)
    B_course = _asm(B_course, ACC_COURSE, 64)
    B_stu_l = _asm(B_stu_l, ACC_STU, 64)
    B_stu_c = _asm(B_stu_c, ACC_STU, 64)

    h2_dorm = _h2_post(B_dorm, deg_d, b2_live, ACC_DORM, interp)
    h2_course = _h2_post(B_course, deg_c, b2_choose, ACC_COURSE, interp)
    stu_out = _stu_final(B_stu_l, B_stu_c, deg_sl, deg_sc,
                         b2_livedby, b2_choosedby,
                         W_ih_f, b_ih_f + b_hh_f, W_ih_b, b_ih_b + b_hh_b,
                         interp)

    return jnp.concatenate(
        [stu_out[:NSTU], h2_dorm[:NDORM], h2_course[:NCOURSE]], axis=0)
